# baseline (speedup 1.0000x reference)
import functools

import jax
import jax.numpy as jnp
from jax.experimental import pallas as pl
from jax.experimental.pallas import tpu as pltpu

_TLHS = (((0,), (0,)), ((), ()))


def _mlp_body(xt_ref, w1t_ref, w2_ref, w3_ref, out_ref):
    h1 = jnp.maximum(
        jax.lax.dot_general(w1t_ref[...], xt_ref[...], _TLHS,
                            preferred_element_type=jnp.float32), 0.0
    )
    h2 = jnp.maximum(
        jnp.dot(w2_ref[...], h1, preferred_element_type=jnp.float32), 0.0
    )
    out_ref[...] = jnp.dot(w3_ref[...], h2,
                           preferred_element_type=jnp.float32)


@functools.partial(jax.jit, static_argnames=("tile_b",))
def _mlp(x, W1, W2, W3, tile_b=2048):
    b, d_in = x.shape
    h = W1.shape[0]
    n_out = W3.shape[0]
    xt = x.T
    w1t = W1.T
    grid = (b // tile_b,)
    out_t = pl.pallas_call(
        _mlp_body,
        grid=grid,
        in_specs=[
            pl.BlockSpec((d_in, tile_b), lambda i: (0, i)),
            pl.BlockSpec((d_in, h), lambda i: (0, 0)),
            pl.BlockSpec((h, h), lambda i: (0, 0)),
            pl.BlockSpec((n_out, h), lambda i: (0, 0)),
        ],
        out_specs=pl.BlockSpec((n_out, tile_b), lambda i: (0, i)),
        out_shape=jax.ShapeDtypeStruct((n_out, b), jnp.float32),
        compiler_params=pltpu.CompilerParams(
            fuse_transposed_lhs_in_matmul=True,
        ),
    )(xt, w1t, W2, W3)
    return out_t.T


def kernel(x, W1, W2, W3):
    return _mlp(x, W1, W2, W3)


# speedup vs baseline: 1.1861x; 1.1861x over previous
import functools

import jax
import jax.numpy as jnp
from jax.experimental import pallas as pl
from jax.experimental.pallas import tpu as pltpu

_TLHS = (((0,), (0,)), ((), ()))


def _mlp_body(xt_ref, w1t_ref, w2_ref, w3_ref, out_ref):
    xb = xt_ref[...].astype(jnp.bfloat16)
    h1 = jnp.maximum(
        jax.lax.dot_general(w1t_ref[...], xb, _TLHS,
                            preferred_element_type=jnp.float32), 0.0
    ).astype(jnp.bfloat16)
    h2 = jnp.maximum(
        jnp.dot(w2_ref[...], h1, preferred_element_type=jnp.float32), 0.0
    ).astype(jnp.bfloat16)
    out_ref[...] = jnp.dot(w3_ref[...], h2,
                           preferred_element_type=jnp.float32)


@functools.partial(jax.jit, static_argnames=("tile_b",))
def _mlp(x, W1, W2, W3, tile_b=2048):
    b, d_in = x.shape
    h = W1.shape[0]
    n_out = W3.shape[0]
    xt = x.T
    w1t = W1.T.astype(jnp.bfloat16)
    W2 = W2.astype(jnp.bfloat16)
    W3 = W3.astype(jnp.bfloat16)
    grid = (b // tile_b,)
    out_t = pl.pallas_call(
        _mlp_body,
        grid=grid,
        in_specs=[
            pl.BlockSpec((d_in, tile_b), lambda i: (0, i)),
            pl.BlockSpec((d_in, h), lambda i: (0, 0)),
            pl.BlockSpec((h, h), lambda i: (0, 0)),
            pl.BlockSpec((n_out, h), lambda i: (0, 0)),
        ],
        out_specs=pl.BlockSpec((n_out, tile_b), lambda i: (0, i)),
        out_shape=jax.ShapeDtypeStruct((n_out, b), jnp.float32),
    )(xt, w1t, W2, W3)
    return out_t.T


def kernel(x, W1, W2, W3):
    return _mlp(x, W1, W2, W3)


# speedup vs baseline: 1.3241x; 1.1163x over previous
import functools

import jax
import jax.numpy as jnp
from jax.experimental import pallas as pl
from jax.experimental.pallas import tpu as pltpu

_TLHS = (((0,), (0,)), ((), ()))


def _mlp_body(xt_ref, w1t_ref, w2_ref, w3_ref, out_ref):
    h1 = jnp.maximum(
        jax.lax.dot_general(w1t_ref[...], xt_ref[...], _TLHS,
                            preferred_element_type=jnp.float32), 0.0
    )
    h2 = jnp.maximum(
        jnp.dot(w2_ref[...], h1, preferred_element_type=jnp.float32), 0.0
    )
    out_ref[...] = jnp.dot(w3_ref[...], h2,
                           preferred_element_type=jnp.float32)


@functools.partial(jax.jit, static_argnames=("tile_b",))
def _mlp(x, W1, W2, W3, tile_b=2048):
    b, d_in = x.shape
    h = W1.shape[0]
    n_out = W3.shape[0]
    xt = x.T
    w1t = W1.T
    grid = (b // tile_b,)
    out_t = pl.pallas_call(
        _mlp_body,
        grid=grid,
        in_specs=[
            pl.BlockSpec((d_in, tile_b), lambda i: (0, i)),
            pl.BlockSpec((d_in, h), lambda i: (0, 0)),
            pl.BlockSpec((h, h), lambda i: (0, 0)),
            pl.BlockSpec((n_out, h), lambda i: (0, 0)),
        ],
        out_specs=pl.BlockSpec((n_out, tile_b), lambda i: (0, i)),
        out_shape=jax.ShapeDtypeStruct((n_out, b), jnp.float32),
    )(xt, w1t, W2, W3)
    return out_t.T


def kernel(x, W1, W2, W3):
    return _mlp(x, W1, W2, W3)


# speedup vs baseline: 1.3277x; 1.0027x over previous
import functools

import jax
import jax.numpy as jnp
from jax.experimental import pallas as pl
from jax.experimental.pallas import tpu as pltpu

_TLHS = (((0,), (0,)), ((), ()))


def _mlp_body(xt_ref, w1t_ref, w2_ref, w3_ref, out_ref):
    h1 = jnp.maximum(
        jax.lax.dot_general(w1t_ref[...], xt_ref[...], _TLHS,
                            preferred_element_type=jnp.float32), 0.0
    )
    h2 = jnp.maximum(
        jnp.dot(w2_ref[...], h1, preferred_element_type=jnp.float32), 0.0
    )
    out_ref[...] = jnp.dot(w3_ref[...], h2,
                           preferred_element_type=jnp.float32)


@functools.partial(jax.jit, static_argnames=("tile_b",))
def _mlp(x, W1, W2, W3, tile_b=2048):
    b, d_in = x.shape
    h = W1.shape[0]
    n_out = W3.shape[0]
    xt = x.T
    w1t = W1.T
    grid = (b // tile_b,)
    out_t = pl.pallas_call(
        _mlp_body,
        grid=grid,
        in_specs=[
            pl.BlockSpec((d_in, tile_b), lambda i: (0, i)),
            pl.BlockSpec((d_in, h), lambda i: (0, 0)),
            pl.BlockSpec((h, h), lambda i: (0, 0)),
            pl.BlockSpec((n_out, h), lambda i: (0, 0)),
        ],
        out_specs=pl.BlockSpec((n_out, tile_b), lambda i: (0, i)),
        out_shape=jax.ShapeDtypeStruct((n_out, b), jnp.float32),
        compiler_params=pltpu.CompilerParams(
            vmem_limit_bytes=100 * 1024 * 1024,
        ),
    )(xt, w1t, W2, W3)
    return out_t.T


def kernel(x, W1, W2, W3):
    return _mlp(x, W1, W2, W3)
